# baseline (speedup 1.0000x reference)
import jax
import jax.numpy as jnp
from jax.experimental import pallas as pl
from jax.experimental.pallas import tpu as pltpu

_N_SAMPLES = 1024
_N_NODES = 256
_N_CHILD = 2048
_BS = 1024


def _body(ll0_ref, ll1_ref, lw0_ref, lw1_ref, out_ref, ew0_ref, ew1_ref, norm_ref):
    @pl.when(pl.program_id(0) == 0)
    def _init():
        ew0 = jnp.exp(lw0_ref[...])
        ew1 = jnp.exp(lw1_ref[...])
        ew0_ref[...] = ew0.astype(jnp.bfloat16)
        ew1_ref[...] = ew1.astype(jnp.bfloat16)
        norm_ref[...] = jnp.log(
            jnp.sum(ew0, axis=1) + jnp.sum(ew1, axis=1)
        )[None, :]

    dn = (((1,), (1,)), ((), ()))
    acc = jax.lax.dot_general(
        jnp.exp(ll0_ref[...]).astype(jnp.bfloat16), ew0_ref[...], dn,
        preferred_element_type=jnp.float32,
    )
    acc = acc + jax.lax.dot_general(
        jnp.exp(ll1_ref[...]).astype(jnp.bfloat16), ew1_ref[...], dn,
        preferred_element_type=jnp.float32,
    )
    out_ref[...] = jnp.log(acc) - norm_ref[...]


def kernel(ll0, ll1, log_weights0, log_weights1):
    n_samples = ll0.shape[0]
    n_nodes = log_weights0.shape[0]
    n_child = ll0.shape[1]
    grid = (n_samples // _BS,)
    return pl.pallas_call(
        _body,
        grid=grid,
        in_specs=[
            pl.BlockSpec((_BS, n_child), lambda i: (i, 0)),
            pl.BlockSpec((_BS, n_child), lambda i: (i, 0)),
            pl.BlockSpec((n_nodes, n_child), lambda i: (0, 0)),
            pl.BlockSpec((n_nodes, n_child), lambda i: (0, 0)),
        ],
        out_specs=pl.BlockSpec((_BS, n_nodes), lambda i: (i, 0)),
        out_shape=jax.ShapeDtypeStruct((n_samples, n_nodes), jnp.float32),
        scratch_shapes=[
            pltpu.VMEM((n_nodes, n_child), jnp.bfloat16),
            pltpu.VMEM((n_nodes, n_child), jnp.bfloat16),
            pltpu.VMEM((1, n_nodes), jnp.float32),
        ],
    )(ll0, ll1, log_weights0, log_weights1)


# speedup vs baseline: 1.2165x; 1.2165x over previous
import jax
import jax.numpy as jnp
from jax.experimental import pallas as pl
from jax.experimental.pallas import tpu as pltpu

_N_SAMPLES = 1024
_N_NODES = 256
_N_CHILD = 2048
_BS = 512


def _body(ll0_ref, ll1_ref, lw0_ref, lw1_ref, out_ref, ew0_ref, ew1_ref, norm_ref):
    @pl.when(pl.program_id(0) == 0)
    def _init():
        ew0 = jnp.exp(lw0_ref[...])
        ew1 = jnp.exp(lw1_ref[...])
        ew0_ref[...] = ew0.astype(jnp.bfloat16)
        ew1_ref[...] = ew1.astype(jnp.bfloat16)
        norm_ref[...] = jnp.log(
            jnp.sum(ew0, axis=1) + jnp.sum(ew1, axis=1)
        )[None, :]

    dn = (((1,), (1,)), ((), ()))
    acc = jax.lax.dot_general(
        jnp.exp(ll0_ref[...]).astype(jnp.bfloat16), ew0_ref[...], dn,
        preferred_element_type=jnp.float32,
    )
    acc = acc + jax.lax.dot_general(
        jnp.exp(ll1_ref[...]).astype(jnp.bfloat16), ew1_ref[...], dn,
        preferred_element_type=jnp.float32,
    )
    out_ref[...] = jnp.log(acc) - norm_ref[...]


def kernel(ll0, ll1, log_weights0, log_weights1):
    n_samples = ll0.shape[0]
    n_nodes = log_weights0.shape[0]
    n_child = ll0.shape[1]
    grid = (n_samples // _BS,)
    return pl.pallas_call(
        _body,
        grid=grid,
        in_specs=[
            pl.BlockSpec((_BS, n_child), lambda i: (i, 0)),
            pl.BlockSpec((_BS, n_child), lambda i: (i, 0)),
            pl.BlockSpec((n_nodes, n_child), lambda i: (0, 0)),
            pl.BlockSpec((n_nodes, n_child), lambda i: (0, 0)),
        ],
        out_specs=pl.BlockSpec((_BS, n_nodes), lambda i: (i, 0)),
        out_shape=jax.ShapeDtypeStruct((n_samples, n_nodes), jnp.float32),
        scratch_shapes=[
            pltpu.VMEM((n_nodes, n_child), jnp.bfloat16),
            pltpu.VMEM((n_nodes, n_child), jnp.bfloat16),
            pltpu.VMEM((1, n_nodes), jnp.float32),
        ],
    )(ll0, ll1, log_weights0, log_weights1)
